# Initial kernel scaffold; baseline (speedup 1.0000x reference)
import jax
import jax.numpy as jnp
from jax.experimental import pallas as pl


def kernel(x, edge_index, edge_attr, node_types, nt_emb, et_emb, Wl0, bl0, Wr0, br0, We0, att0, bias0, Wl1, bl1, Wr1, br1, We1, att1, bias1, Wo, bo):
    raise NotImplementedError("write your pallas kernel here")


# speedup vs baseline: 16.6215x; 16.6215x over previous
import functools

import jax
import jax.numpy as jnp
from jax import lax
from jax.experimental import pallas as pl
from jax.experimental.pallas import tpu as pltpu
from jax.experimental.pallas import tpu_sc as plsc

_NC = 2
_NS = 16
_NW = _NC * _NS
_CH = 64


def _h0_body(t_ref, x_ref, nt_ref, h_ref):
    t = t_ref[...]
    h = x_ref[...] + jnp.broadcast_to(nt_ref[0:1, :], x_ref.shape)
    for k in range(1, 5):
        h = jnp.where(t == k, x_ref[...] + nt_ref[k:k + 1, :], h)
    h_ref[...] = h


def _counts_body(a_ref, et_ref, tab_ref, cnt_ref, *, nsteps, e_total):
    i = pl.program_id(0)

    @pl.when(i == 0)
    def _():
        for c in range(3):
            cnt_ref[c] = 0.0

    blk = a_ref[...]
    for c in range(3):
        cnt_ref[c] += jnp.sum((blk == c).astype(jnp.float32))

    @pl.when(i == nsteps - 1)
    def _():
        inv = 1.0 / e_total
        mean = (cnt_ref[0] * et_ref[0:1, :] + cnt_ref[1] * et_ref[1:2, :]
                + cnt_ref[2] * et_ref[2:3, :]) * inv
        tab_ref[...] = jnp.concatenate(
            [et_ref[...], mean, jnp.zeros((4, 128), jnp.float32)], axis=0)


def _tabproj_body(tab_ref, we_ref, out_ref):
    out_ref[...] = lax.dot_general(
        tab_ref[...], we_ref[...], (((1,), (1,)), ((), ())),
        preferred_element_type=jnp.float32)


def _proj_body(h_ref, wl_ref, bl_ref, wr_ref, br_ref, xl_ref, xr_ref):
    h = h_ref[...]
    xl_ref[...] = lax.dot_general(
        h, wl_ref[...], (((1,), (1,)), ((), ())),
        preferred_element_type=jnp.float32) + bl_ref[...]
    xr_ref[...] = lax.dot_general(
        h, wr_ref[...], (((1,), (1,)), ((), ())),
        preferred_element_type=jnp.float32) + br_ref[...]


def _epi_body(h_ref, xl_ref, xr_ref, acc_ref, dn_ref, proj_ref, attf_ref,
              seg_ref, x16_ref, bias_ref, *rest, final):
    if final:
        wo_ref, bo_ref, out_ref = rest
    else:
        (out_ref,) = rest
    xl = xl_ref[...]
    acc = acc_ref[0] + acc_ref[1]
    dn16 = dn_ref[0] + dn_ref[1]
    s = xl + xr_ref[...] + proj_ref[3:4, :]
    z = jnp.where(s >= 0, s, 0.2 * s)
    alpha128 = lax.dot_general(
        z * attf_ref[...], seg_ref[...], (((1,), (0,)), ((), ())),
        preferred_element_type=jnp.float32)
    ex128 = jnp.exp(alpha128)
    dn128 = lax.dot_general(
        dn16, x16_ref[...], (((1,), (0,)), ((), ())),
        preferred_element_type=jnp.float32)
    g = (acc + xl * ex128) / (dn128 + ex128)
    hn = jnp.maximum(g + bias_ref[...], 0.0) + h_ref[...]
    if final:
        out_ref[...] = lax.dot_general(
            hn, wo_ref[...], (((1,), (1,)), ((), ())),
            preferred_element_type=jnp.float32) + bo_ref[...]
    else:
        out_ref[...] = hn


def _row_chunks(rows):
    nfull, rem = divmod(rows, _CH)
    sizes = [_CH] * nfull + ([rem] if rem else [])
    offs, o = [], 0
    for s in sizes:
        offs.append(o)
        o += s
    return list(zip(offs, sizes))


def _sc_body(src_hbm, dstg_hbm, dsts_hbm, attr_hbm, xl_hbm, xr_hbm, tab_hbm,
             att_hbm, accp_hbm, dnp_hbm,
             srcb, dstgb, dstsb, dstdb, attrb, xlb, xrb, wb, dnb, tab, attv,
             acc_sh, dn_sh, sem1, sem2, *, kc, n_acc):
    cid = lax.axis_index("c")
    sid = lax.axis_index("s")
    wid = sid * _NC + cid
    rows = n_acc // _NS
    rows_d = (n_acc // 8) // _NS
    r0 = sid * rows
    r0d = sid * rows_d

    def zrow_body(i, carry):
        for kk in range(8):
            xlb[i, pl.ds(16 * kk, 16)] = jnp.zeros((16,), jnp.float32)
        return carry

    lax.fori_loop(0, _CH, zrow_body, 0)
    for off, sz in _row_chunks(rows):
        pltpu.sync_copy(xlb.at[pl.ds(0, sz)], acc_sh.at[pl.ds(r0 + off, sz)])
    for off, sz in _row_chunks(rows_d):
        pltpu.sync_copy(xlb.at[pl.ds(0, sz)], dn_sh.at[pl.ds(r0d + off, sz)])
    pltpu.sync_copy(tab_hbm, tab)
    pltpu.sync_copy(att_hbm, attv)
    plsc.subcore_barrier()

    col = lax.iota(jnp.int32, 16)
    att_regs = [attv[0, pl.ds(16 * h, 16)] for h in range(8)]
    base0 = wid * (kc * _CH)

    def chunk_body(k, carry):
        base = base0 + k * _CH
        pltpu.sync_copy(src_hbm.at[pl.ds(base, _CH)], srcb)
        pltpu.sync_copy(dstg_hbm.at[pl.ds(base, _CH)], dstgb)
        pltpu.sync_copy(dsts_hbm.at[pl.ds(base, _CH)], dstsb)
        pltpu.sync_copy(attr_hbm.at[pl.ds(base, _CH)], attrb)
        c1 = pltpu.async_copy(xl_hbm.at[srcb], xlb, sem1)
        c2 = pltpu.async_copy(xr_hbm.at[dstgb], xrb, sem2)
        c1.wait()
        c2.wait()

        for g in range(_CH // 16):
            dstdb[pl.ds(16 * g, 16)] = lax.shift_right_logical(
                dstsb[pl.ds(16 * g, 16)], 3)

        def edge_body(e, c2_):
            efull = jnp.full((16,), e, jnp.int32)
            a_splat = plsc.load_gather(attrb, [efull])
            j_splat = jnp.bitwise_and(plsc.load_gather(dstsb, [efull]), 7)
            rowb = a_splat * 128 + col
            dnv = jnp.zeros((16,), jnp.float32)
            for h in range(8):
                xlv = xlb[e, pl.ds(16 * h, 16)]
                xrv = xrb[e, pl.ds(16 * h, 16)]
                eev = plsc.load_gather(tab, [rowb + 16 * h])
                s = xlv + xrv + eev
                z = jnp.where(s >= 0, s, 0.2 * s)
                alpha = jnp.sum(z * att_regs[h])
                exv = jnp.exp(jnp.full((16,), alpha))
                wb[e, pl.ds(16 * h, 16)] = xlv * exv
                dnv = jnp.where(col == h, exv, dnv)
            zero = jnp.zeros((16,), jnp.float32)
            for j in range(8):
                dnb[e, pl.ds(16 * j, 16)] = jnp.where(j_splat == j, dnv, zero)
            return c2_

        lax.fori_loop(0, _CH, edge_body, 0)
        pltpu.sync_copy(wb, acc_sh.at[dstsb], add=True)
        pltpu.sync_copy(dnb, dn_sh.at[dstdb], add=True)
        return carry

    lax.fori_loop(0, kc, chunk_body, 0)
    plsc.subcore_barrier()
    for off, sz in _row_chunks(rows):
        pltpu.sync_copy(acc_sh.at[pl.ds(r0 + off, sz)], xlb.at[pl.ds(0, sz)])
        pltpu.sync_copy(xlb.at[pl.ds(0, sz)],
                        accp_hbm.at[cid, pl.ds(r0 + off, sz)])
    for off, sz in _row_chunks(rows_d):
        pltpu.sync_copy(dn_sh.at[pl.ds(r0d + off, sz)], xlb.at[pl.ds(0, sz)])
        pltpu.sync_copy(xlb.at[pl.ds(0, sz)],
                        dnp_hbm.at[cid, pl.ds(r0d + off, sz)])


def kernel(x, edge_index, edge_attr, node_types, nt_emb, et_emb,
           Wl0, bl0, Wr0, br0, We0, att0, bias0,
           Wl1, bl1, Wr1, br1, We1, att1, bias1,
           Wo, bo):
    n, d = x.shape
    e_num = edge_attr.shape[0]
    f32 = jnp.float32

    kc = -(-e_num // (_NW * _CH))
    e_pad = _NW * _CH * kc
    pad = e_pad - e_num
    idt = edge_index.dtype
    src = jnp.concatenate([edge_index[0], jnp.zeros((pad,), idt)])
    dstg = jnp.concatenate([edge_index[1], jnp.zeros((pad,), idt)])
    dsts = jnp.concatenate([edge_index[1], jnp.full((pad,), n, idt)])
    attr = jnp.concatenate([edge_attr.astype(idt), jnp.zeros((pad,), idt)])
    n_acc = -(-(n + 1) // 1024) * 1024

    jidx = jnp.arange(128) // 16
    seg = (jidx[:, None] == jidx[None, :]).astype(f32)
    x16 = (jnp.arange(16)[:, None] == jidx[None, :]).astype(f32)

    r = 1000
    gn = n // r

    h0 = pl.pallas_call(
        _h0_body,
        grid=(gn,),
        in_specs=[pl.BlockSpec((r, 1), lambda i: (i, 0)),
                  pl.BlockSpec((r, 128), lambda i: (i, 0)),
                  pl.BlockSpec((5, 128), lambda i: (0, 0))],
        out_specs=pl.BlockSpec((r, 128), lambda i: (i, 0)),
        out_shape=jax.ShapeDtypeStruct((n, 128), f32),
    )(node_types.reshape(n, 1), x, nt_emb)

    eb = e_num // 128
    ebs = eb
    nsteps = 1
    tab8 = pl.pallas_call(
        functools.partial(_counts_body, nsteps=nsteps, e_total=float(e_num)),
        grid=(nsteps,),
        in_specs=[pl.BlockSpec((ebs, 128), lambda i: (i, 0)),
                  pl.BlockSpec((3, 128), lambda i: (0, 0))],
        out_specs=pl.BlockSpec((8, 128), lambda i: (0, 0)),
        out_shape=jax.ShapeDtypeStruct((8, 128), f32),
        scratch_shapes=[pltpu.SMEM((3,), f32)],
    )(edge_attr.reshape(eb, 128), et_emb)

    proj_call = pl.pallas_call(
        _proj_body,
        grid=(gn,),
        in_specs=[pl.BlockSpec((r, 128), lambda i: (i, 0)),
                  pl.BlockSpec((128, 128), lambda i: (0, 0)),
                  pl.BlockSpec((1, 128), lambda i: (0, 0)),
                  pl.BlockSpec((128, 128), lambda i: (0, 0)),
                  pl.BlockSpec((1, 128), lambda i: (0, 0))],
        out_specs=[pl.BlockSpec((r, 128), lambda i: (i, 0)),
                   pl.BlockSpec((r, 128), lambda i: (i, 0))],
        out_shape=[jax.ShapeDtypeStruct((n, 128), f32),
                   jax.ShapeDtypeStruct((n, 128), f32)],
    )

    tabproj_call = pl.pallas_call(
        _tabproj_body,
        out_shape=jax.ShapeDtypeStruct((8, 128), f32),
    )

    def epi_call(final, nout):
        extra = ([pl.BlockSpec((128, 128), lambda i: (0, 0)),
                  pl.BlockSpec((1, 128), lambda i: (0, 0))] if final else [])
        return pl.pallas_call(
            functools.partial(_epi_body, final=final),
            grid=(gn,),
            in_specs=[pl.BlockSpec((r, 128), lambda i: (i, 0)),
                      pl.BlockSpec((r, 128), lambda i: (i, 0)),
                      pl.BlockSpec((r, 128), lambda i: (i, 0)),
                      pl.BlockSpec((2, r, 128), lambda i: (0, i, 0)),
                      pl.BlockSpec((2, r, 16), lambda i: (0, i, 0)),
                      pl.BlockSpec((8, 128), lambda i: (0, 0)),
                      pl.BlockSpec((1, 128), lambda i: (0, 0)),
                      pl.BlockSpec((128, 128), lambda i: (0, 0)),
                      pl.BlockSpec((16, 128), lambda i: (0, 0)),
                      pl.BlockSpec((1, 128), lambda i: (0, 0))] + extra,
            out_specs=pl.BlockSpec((r, 128), lambda i: (i, 0)),
            out_shape=jax.ShapeDtypeStruct((nout, 128), f32),
        )

    h = h0
    layers = [(Wl0, bl0, Wr0, br0, We0, att0, bias0),
              (Wl1, bl1, Wr1, br1, We1, att1, bias1)]
    for li, (wl, bl, wr, br, we, att, bias) in enumerate(layers):
        proj8 = tabproj_call(tab8, we)
        xl, xr = proj_call(h, wl, bl.reshape(1, 128), wr, br.reshape(1, 128))
        accp, dnp = _sc_edge_pass(src, dstg, dsts, attr, xl, xr, proj8, att,
                                  kc=kc, n_acc=n_acc)
        final = li == 1
        args = [h, xl, xr, accp, dnp, proj8, att.reshape(1, 128), seg, x16,
                bias.reshape(1, 128)]
        if final:
            args += [Wo, bo.reshape(1, 128)]
        h = epi_call(final, n)(*args)
    return h


def _sc_edge_pass(src, dstg, dsts, attr, xl, xr, proj8, att, *, kc, n_acc):
    f32 = jnp.float32
    mesh = plsc.VectorSubcoreMesh(core_axis_name="c", subcore_axis_name="s",
                                  num_cores=_NC, num_subcores=_NS)
    sc_call = pl.kernel(
        functools.partial(_sc_body, kc=kc, n_acc=n_acc),
        out_type=(jax.ShapeDtypeStruct((_NC, n_acc, 128), f32),
                  jax.ShapeDtypeStruct((_NC, n_acc // 8, 128), f32)),
        mesh=mesh,
        compiler_params=pltpu.CompilerParams(needs_layout_passes=False),
        scratch_types=[
            pltpu.VMEM((_CH,), jnp.int32),
            pltpu.VMEM((_CH,), jnp.int32),
            pltpu.VMEM((_CH,), jnp.int32),
            pltpu.VMEM((_CH,), jnp.int32),
            pltpu.VMEM((_CH,), jnp.int32),
            pltpu.VMEM((_CH, 128), f32),
            pltpu.VMEM((_CH, 128), f32),
            pltpu.VMEM((_CH, 128), f32),
            pltpu.VMEM((_CH, 128), f32),
            pltpu.VMEM((1024,), f32),
            pltpu.VMEM((1, 128), f32),
            pltpu.MemorySpace.VMEM_SHARED((n_acc, 128), f32),
            pltpu.MemorySpace.VMEM_SHARED((n_acc // 8, 128), f32),
            pltpu.SemaphoreType.DMA,
            pltpu.SemaphoreType.DMA,
        ],
    )
    accp, dnp = sc_call(src, dstg, dsts, attr, xl, xr, proj8.reshape(-1),
                        att.reshape(1, 128))
    return accp, dnp.reshape(_NC, n_acc, 16)


# speedup vs baseline: 18.3555x; 1.1043x over previous
import functools

import jax
import jax.numpy as jnp
from jax import lax
from jax.experimental import pallas as pl
from jax.experimental.pallas import tpu as pltpu
from jax.experimental.pallas import tpu_sc as plsc

_NC = 2
_NS = 16
_NW = _NC * _NS
_CH = 64


def _h0_body(t_ref, x_ref, nt_ref, h_ref):
    t = t_ref[...]
    h = x_ref[...] + jnp.broadcast_to(nt_ref[0:1, :], x_ref.shape)
    for k in range(1, 5):
        h = jnp.where(t == k, x_ref[...] + nt_ref[k:k + 1, :], h)
    h_ref[...] = h


def _counts_body(a_ref, et_ref, tab_ref, cnt_ref, *, nsteps, e_total):
    i = pl.program_id(0)

    @pl.when(i == 0)
    def _():
        for c in range(3):
            cnt_ref[c] = 0.0

    blk = a_ref[...]
    for c in range(3):
        cnt_ref[c] += jnp.sum((blk == c).astype(jnp.float32))

    @pl.when(i == nsteps - 1)
    def _():
        inv = 1.0 / e_total
        mean = (cnt_ref[0] * et_ref[0:1, :] + cnt_ref[1] * et_ref[1:2, :]
                + cnt_ref[2] * et_ref[2:3, :]) * inv
        tab_ref[...] = jnp.concatenate(
            [et_ref[...], mean, jnp.zeros((4, 128), jnp.float32)], axis=0)


def _tabproj_body(tab_ref, we_ref, out_ref):
    out_ref[...] = lax.dot_general(
        tab_ref[...], we_ref[...], (((1,), (1,)), ((), ())),
        preferred_element_type=jnp.float32)


def _proj_body(h_ref, wl_ref, bl_ref, wr_ref, br_ref, xl_ref, xr_ref):
    h = h_ref[...]
    xl_ref[...] = lax.dot_general(
        h, wl_ref[...], (((1,), (1,)), ((), ())),
        preferred_element_type=jnp.float32) + bl_ref[...]
    xr_ref[...] = lax.dot_general(
        h, wr_ref[...], (((1,), (1,)), ((), ())),
        preferred_element_type=jnp.float32) + br_ref[...]


def _epi_body(h_ref, xl_ref, xr_ref, acc_ref, dn_ref, proj_ref, attf_ref,
              seg_ref, x16_ref, bias_ref, *rest, final):
    if final:
        wo_ref, bo_ref, out_ref = rest
    else:
        (out_ref,) = rest
    xl = xl_ref[...]
    acc = acc_ref[0] + acc_ref[1]
    dn16 = dn_ref[0] + dn_ref[1]
    s = xl + xr_ref[...] + proj_ref[3:4, :]
    z = jnp.where(s >= 0, s, 0.2 * s)
    alpha128 = lax.dot_general(
        z * attf_ref[...], seg_ref[...], (((1,), (0,)), ((), ())),
        preferred_element_type=jnp.float32)
    ex128 = jnp.exp(alpha128)
    dn128 = lax.dot_general(
        dn16, x16_ref[...], (((1,), (0,)), ((), ())),
        preferred_element_type=jnp.float32)
    g = (acc + xl * ex128) / (dn128 + ex128)
    hn = jnp.maximum(g + bias_ref[...], 0.0) + h_ref[...]
    if final:
        out_ref[...] = lax.dot_general(
            hn, wo_ref[...], (((1,), (1,)), ((), ())),
            preferred_element_type=jnp.float32) + bo_ref[...]
    else:
        out_ref[...] = hn


def _row_chunks(rows):
    nfull, rem = divmod(rows, _CH)
    sizes = [_CH] * nfull + ([rem] if rem else [])
    offs, o = [], 0
    for s in sizes:
        offs.append(o)
        o += s
    return list(zip(offs, sizes))


def _sc_body(idxp_hbm, xl_hbm, xr_hbm, tab_hbm,
             att_hbm, accp_hbm, dnp_hbm,
             idxb, dstsb, dstdb, xlb, xrb, wb, dnb, tab, attv,
             acc_sh, dn_sh, sem1, sem2, semw, semd, *, kc, n_acc):
    cid = lax.axis_index("c")
    sid = lax.axis_index("s")
    wid = sid * _NC + cid
    rows = n_acc // _NS
    rows_d = (n_acc // 8) // _NS
    r0 = sid * rows
    r0d = sid * rows_d

    def zrow_body(i, carry):
        for kk in range(8):
            xlb[i, pl.ds(16 * kk, 16)] = jnp.zeros((16,), jnp.float32)
        return carry

    lax.fori_loop(0, _CH, zrow_body, 0)
    for off, sz in _row_chunks(rows):
        pltpu.sync_copy(xlb.at[pl.ds(0, sz)], acc_sh.at[pl.ds(r0 + off, sz)])
    for off, sz in _row_chunks(rows_d):
        pltpu.sync_copy(xlb.at[pl.ds(0, sz)], dn_sh.at[pl.ds(r0d + off, sz)])
    pltpu.sync_copy(tab_hbm, tab)
    pltpu.sync_copy(att_hbm, attv)
    plsc.subcore_barrier()

    col = lax.iota(jnp.int32, 16)
    att_regs = [attv[0, pl.ds(16 * h, 16)] for h in range(8)]
    base0 = wid * kc

    def chunk_body(k, carry):
        base = (base0 + k) * (4 * _CH)
        pltpu.sync_copy(idxp_hbm.at[pl.ds(base, 4 * _CH)], idxb)
        c1 = pltpu.async_copy(xl_hbm.at[idxb.at[pl.ds(0, _CH)]], xlb, sem1)
        c2 = pltpu.async_copy(xr_hbm.at[idxb.at[pl.ds(_CH, _CH)]], xrb, sem2)

        @pl.when(k > 0)
        def _():
            pltpu.make_async_copy(wb, acc_sh.at[dstsb], semw).wait()
            pltpu.make_async_copy(dnb, dn_sh.at[dstdb], semd).wait()

        for g in range(_CH // 16):
            v = idxb[pl.ds(2 * _CH + 16 * g, 16)]
            dstsb[pl.ds(16 * g, 16)] = v
            dstdb[pl.ds(16 * g, 16)] = lax.shift_right_logical(v, 3)
        c1.wait()
        c2.wait()

        def edge_body(e, carry_):
            efull = jnp.full((16,), e, jnp.int32)
            a_splat = plsc.load_gather(idxb, [efull + 3 * _CH])
            j_splat = jnp.bitwise_and(
                plsc.load_gather(idxb, [efull + 2 * _CH]), 7)
            rowb = a_splat * 128 + col
            dnv = jnp.zeros((16,), jnp.float32)
            for h in range(8):
                xlv = xlb[e, pl.ds(16 * h, 16)]
                xrv = xrb[e, pl.ds(16 * h, 16)]
                eev = plsc.load_gather(tab, [rowb + 16 * h])
                s = xlv + xrv + eev
                z = jnp.where(s >= 0, s, 0.2 * s)
                alpha = jnp.sum(z * att_regs[h])
                exv = jnp.exp(jnp.full((16,), alpha))
                wb[e, pl.ds(16 * h, 16)] = xlv * exv
                dnv = jnp.where(col == h, exv, dnv)
            zero = jnp.zeros((16,), jnp.float32)
            for j in range(8):
                dnb[e, pl.ds(16 * j, 16)] = jnp.where(j_splat == j, dnv, zero)
            return carry_

        lax.fori_loop(0, _CH, edge_body, 0)
        pltpu.async_copy(wb, acc_sh.at[dstsb], semw, add=True)
        pltpu.async_copy(dnb, dn_sh.at[dstdb], semd, add=True)
        return carry

    lax.fori_loop(0, kc, chunk_body, 0)
    pltpu.make_async_copy(wb, acc_sh.at[dstsb], semw).wait()
    pltpu.make_async_copy(dnb, dn_sh.at[dstdb], semd).wait()
    plsc.subcore_barrier()
    for off, sz in _row_chunks(rows):
        pltpu.sync_copy(acc_sh.at[pl.ds(r0 + off, sz)], xlb.at[pl.ds(0, sz)])
        pltpu.sync_copy(xlb.at[pl.ds(0, sz)],
                        accp_hbm.at[cid, pl.ds(r0 + off, sz)])
    for off, sz in _row_chunks(rows_d):
        pltpu.sync_copy(dn_sh.at[pl.ds(r0d + off, sz)], xlb.at[pl.ds(0, sz)])
        pltpu.sync_copy(xlb.at[pl.ds(0, sz)],
                        dnp_hbm.at[cid, pl.ds(r0d + off, sz)])


def kernel(x, edge_index, edge_attr, node_types, nt_emb, et_emb,
           Wl0, bl0, Wr0, br0, We0, att0, bias0,
           Wl1, bl1, Wr1, br1, We1, att1, bias1,
           Wo, bo):
    n, d = x.shape
    e_num = edge_attr.shape[0]
    f32 = jnp.float32

    kc = -(-e_num // (_NW * _CH))
    e_pad = _NW * _CH * kc
    pad = e_pad - e_num
    idt = edge_index.dtype
    src = jnp.concatenate([edge_index[0], jnp.zeros((pad,), idt)])
    dstg = jnp.concatenate([edge_index[1], jnp.zeros((pad,), idt)])
    dsts = jnp.concatenate([edge_index[1], jnp.full((pad,), n, idt)])
    attr = jnp.concatenate([edge_attr.astype(idt), jnp.zeros((pad,), idt)])
    idxp = (jnp.stack([src, dstg, dsts, attr])
            .reshape(4, _NW * kc, _CH)
            .transpose(1, 0, 2)
            .reshape(-1))
    n_acc = -(-(n + 1) // 1024) * 1024

    jidx = jnp.arange(128) // 16
    seg = (jidx[:, None] == jidx[None, :]).astype(f32)
    x16 = (jnp.arange(16)[:, None] == jidx[None, :]).astype(f32)

    r = 1000
    gn = n // r

    h0 = pl.pallas_call(
        _h0_body,
        grid=(gn,),
        in_specs=[pl.BlockSpec((r, 1), lambda i: (i, 0)),
                  pl.BlockSpec((r, 128), lambda i: (i, 0)),
                  pl.BlockSpec((5, 128), lambda i: (0, 0))],
        out_specs=pl.BlockSpec((r, 128), lambda i: (i, 0)),
        out_shape=jax.ShapeDtypeStruct((n, 128), f32),
    )(node_types.reshape(n, 1), x, nt_emb)

    eb = e_num // 128
    ebs = eb
    nsteps = 1
    tab8 = pl.pallas_call(
        functools.partial(_counts_body, nsteps=nsteps, e_total=float(e_num)),
        grid=(nsteps,),
        in_specs=[pl.BlockSpec((ebs, 128), lambda i: (i, 0)),
                  pl.BlockSpec((3, 128), lambda i: (0, 0))],
        out_specs=pl.BlockSpec((8, 128), lambda i: (0, 0)),
        out_shape=jax.ShapeDtypeStruct((8, 128), f32),
        scratch_shapes=[pltpu.SMEM((3,), f32)],
    )(edge_attr.reshape(eb, 128), et_emb)

    proj_call = pl.pallas_call(
        _proj_body,
        grid=(gn,),
        in_specs=[pl.BlockSpec((r, 128), lambda i: (i, 0)),
                  pl.BlockSpec((128, 128), lambda i: (0, 0)),
                  pl.BlockSpec((1, 128), lambda i: (0, 0)),
                  pl.BlockSpec((128, 128), lambda i: (0, 0)),
                  pl.BlockSpec((1, 128), lambda i: (0, 0))],
        out_specs=[pl.BlockSpec((r, 128), lambda i: (i, 0)),
                   pl.BlockSpec((r, 128), lambda i: (i, 0))],
        out_shape=[jax.ShapeDtypeStruct((n, 128), f32),
                   jax.ShapeDtypeStruct((n, 128), f32)],
    )

    tabproj_call = pl.pallas_call(
        _tabproj_body,
        out_shape=jax.ShapeDtypeStruct((8, 128), f32),
    )

    def epi_call(final, nout):
        extra = ([pl.BlockSpec((128, 128), lambda i: (0, 0)),
                  pl.BlockSpec((1, 128), lambda i: (0, 0))] if final else [])
        return pl.pallas_call(
            functools.partial(_epi_body, final=final),
            grid=(gn,),
            in_specs=[pl.BlockSpec((r, 128), lambda i: (i, 0)),
                      pl.BlockSpec((r, 128), lambda i: (i, 0)),
                      pl.BlockSpec((r, 128), lambda i: (i, 0)),
                      pl.BlockSpec((2, r, 128), lambda i: (0, i, 0)),
                      pl.BlockSpec((2, r, 16), lambda i: (0, i, 0)),
                      pl.BlockSpec((8, 128), lambda i: (0, 0)),
                      pl.BlockSpec((1, 128), lambda i: (0, 0)),
                      pl.BlockSpec((128, 128), lambda i: (0, 0)),
                      pl.BlockSpec((16, 128), lambda i: (0, 0)),
                      pl.BlockSpec((1, 128), lambda i: (0, 0))] + extra,
            out_specs=pl.BlockSpec((r, 128), lambda i: (i, 0)),
            out_shape=jax.ShapeDtypeStruct((nout, 128), f32),
        )

    h = h0
    layers = [(Wl0, bl0, Wr0, br0, We0, att0, bias0),
              (Wl1, bl1, Wr1, br1, We1, att1, bias1)]
    for li, (wl, bl, wr, br, we, att, bias) in enumerate(layers):
        proj8 = tabproj_call(tab8, we)
        xl, xr = proj_call(h, wl, bl.reshape(1, 128), wr, br.reshape(1, 128))
        accp, dnp = _sc_edge_pass(idxp, xl, xr, proj8, att,
                                  kc=kc, n_acc=n_acc)
        final = li == 1
        args = [h, xl, xr, accp, dnp, proj8, att.reshape(1, 128), seg, x16,
                bias.reshape(1, 128)]
        if final:
            args += [Wo, bo.reshape(1, 128)]
        h = epi_call(final, n)(*args)
    return h


def _sc_edge_pass(idxp, xl, xr, proj8, att, *, kc, n_acc):
    f32 = jnp.float32
    mesh = plsc.VectorSubcoreMesh(core_axis_name="c", subcore_axis_name="s",
                                  num_cores=_NC, num_subcores=_NS)
    sc_call = pl.kernel(
        functools.partial(_sc_body, kc=kc, n_acc=n_acc),
        out_type=(jax.ShapeDtypeStruct((_NC, n_acc, 128), f32),
                  jax.ShapeDtypeStruct((_NC, n_acc // 8, 128), f32)),
        mesh=mesh,
        compiler_params=pltpu.CompilerParams(needs_layout_passes=False),
        scratch_types=[
            pltpu.VMEM((4 * _CH,), jnp.int32),
            pltpu.VMEM((_CH,), jnp.int32),
            pltpu.VMEM((_CH,), jnp.int32),
            pltpu.VMEM((_CH, 128), f32),
            pltpu.VMEM((_CH, 128), f32),
            pltpu.VMEM((_CH, 128), f32),
            pltpu.VMEM((_CH, 128), f32),
            pltpu.VMEM((1024,), f32),
            pltpu.VMEM((1, 128), f32),
            pltpu.MemorySpace.VMEM_SHARED((n_acc, 128), f32),
            pltpu.MemorySpace.VMEM_SHARED((n_acc // 8, 128), f32),
            pltpu.SemaphoreType.DMA,
            pltpu.SemaphoreType.DMA,
            pltpu.SemaphoreType.DMA,
            pltpu.SemaphoreType.DMA,
        ],
    )
    accp, dnp = sc_call(idxp, xl, xr, proj8.reshape(-1), att.reshape(1, 128))
    return accp, dnp.reshape(_NC, n_acc, 16)


# speedup vs baseline: 19.9068x; 1.0845x over previous
import functools

import jax
import jax.numpy as jnp
from jax import lax
from jax.experimental import pallas as pl
from jax.experimental.pallas import tpu as pltpu
from jax.experimental.pallas import tpu_sc as plsc

_NC = 2
_NS = 16
_NW = _NC * _NS
_CH = 48


def _h0_body(t_ref, x_ref, nt_ref, h_ref):
    t = t_ref[...]
    h = x_ref[...] + jnp.broadcast_to(nt_ref[0:1, :], x_ref.shape)
    for k in range(1, 5):
        h = jnp.where(t == k, x_ref[...] + nt_ref[k:k + 1, :], h)
    h_ref[...] = h


def _counts_body(a_ref, et_ref, tab_ref, cnt_ref, *, nsteps, e_total):
    i = pl.program_id(0)

    @pl.when(i == 0)
    def _():
        for c in range(3):
            cnt_ref[c] = 0.0

    blk = a_ref[...]
    for c in range(3):
        cnt_ref[c] += jnp.sum((blk == c).astype(jnp.float32))

    @pl.when(i == nsteps - 1)
    def _():
        inv = 1.0 / e_total
        mean = (cnt_ref[0] * et_ref[0:1, :] + cnt_ref[1] * et_ref[1:2, :]
                + cnt_ref[2] * et_ref[2:3, :]) * inv
        tab_ref[...] = jnp.concatenate(
            [et_ref[...], mean, jnp.zeros((4, 128), jnp.float32)], axis=0)


def _tabproj_body(tab_ref, we_ref, out_ref):
    out_ref[...] = lax.dot_general(
        tab_ref[...], we_ref[...], (((1,), (1,)), ((), ())),
        preferred_element_type=jnp.float32)


def _proj_body(h_ref, wl_ref, bl_ref, wr_ref, br_ref, xl_ref, xr_ref):
    h = h_ref[...]
    xl_ref[...] = lax.dot_general(
        h, wl_ref[...], (((1,), (1,)), ((), ())),
        preferred_element_type=jnp.float32) + bl_ref[...]
    xr_ref[...] = lax.dot_general(
        h, wr_ref[...], (((1,), (1,)), ((), ())),
        preferred_element_type=jnp.float32) + br_ref[...]


def _epi_body(h_ref, xl_ref, xr_ref, acc_ref, dn_ref, proj_ref, attf_ref,
              seg_ref, x16_ref, bias_ref, *rest, final):
    if final:
        wo_ref, bo_ref, out_ref = rest
    else:
        (out_ref,) = rest
    xl = xl_ref[...]
    acc = acc_ref[0] + acc_ref[1]
    dn16 = dn_ref[0] + dn_ref[1]
    s = xl + xr_ref[...] + proj_ref[3:4, :]
    z = jnp.where(s >= 0, s, 0.2 * s)
    alpha128 = lax.dot_general(
        z * attf_ref[...], seg_ref[...], (((1,), (0,)), ((), ())),
        preferred_element_type=jnp.float32)
    ex128 = jnp.exp(alpha128)
    dn128 = lax.dot_general(
        dn16, x16_ref[...], (((1,), (0,)), ((), ())),
        preferred_element_type=jnp.float32)
    g = (acc + xl * ex128) / (dn128 + ex128)
    hn = jnp.maximum(g + bias_ref[...], 0.0) + h_ref[...]
    if final:
        out_ref[...] = lax.dot_general(
            hn, wo_ref[...], (((1,), (1,)), ((), ())),
            preferred_element_type=jnp.float32) + bo_ref[...]
    else:
        out_ref[...] = hn


def _row_chunks(rows):
    nfull, rem = divmod(rows, _CH)
    sizes = [_CH] * nfull + ([rem] if rem else [])
    offs, o = [], 0
    for s in sizes:
        offs.append(o)
        o += s
    return list(zip(offs, sizes))


def _sc_body(idxp_hbm, xl_hbm, xr_hbm, tab_hbm,
             att_hbm, accp_hbm, dnp_hbm,
             idxb0, idxb1, dstsb, dstdb, xlb0, xlb1, xrb0, xrb1, wb, dnb,
             tab, attv, acc_sh, dn_sh,
             si0, si1, sl0, sl1, sr0, sr1, semw, semd, *, kc, n_acc, n_dn):
    cid = lax.axis_index("c")
    sid = lax.axis_index("s")
    wid = sid * _NC + cid
    rows = n_acc // _NS
    rows_d = n_dn // _NS
    r0 = sid * rows
    r0d = sid * rows_d

    def zrow_body(i, carry):
        for kk in range(8):
            wb[i, pl.ds(16 * kk, 16)] = jnp.zeros((16,), jnp.float32)
        return carry

    lax.fori_loop(0, _CH, zrow_body, 0)
    for off, sz in _row_chunks(rows):
        pltpu.sync_copy(wb.at[pl.ds(0, sz)], acc_sh.at[pl.ds(r0 + off, sz)])
    for off, sz in _row_chunks(rows_d):
        pltpu.sync_copy(wb.at[pl.ds(0, sz)], dn_sh.at[pl.ds(r0d + off, sz)])
    pltpu.sync_copy(tab_hbm, tab)
    pltpu.sync_copy(att_hbm, attv)
    plsc.subcore_barrier()

    col = lax.iota(jnp.int32, 16)
    att_regs = [attv[0, pl.ds(16 * h, 16)] for h in range(8)]
    base0 = wid * kc

    def idx_slice(ib, part):
        return ib.at[pl.ds(part * _CH, _CH)]

    pltpu.sync_copy(idxp_hbm.at[pl.ds(base0 * 4 * _CH, 4 * _CH)], idxb0)
    pltpu.async_copy(xl_hbm.at[idx_slice(idxb0, 0)], xlb0, sl0)
    pltpu.async_copy(xr_hbm.at[idx_slice(idxb0, 1)], xrb0, sr0)

    def process(k, first, ib, xb, rb, ssl, ssr, nib, nxb, nrb, nsi, nsl, nsr):
        @pl.when(k + 1 < kc)
        def _():
            nbase = (base0 + k + 1) * (4 * _CH)
            pltpu.async_copy(idxp_hbm.at[pl.ds(nbase, 4 * _CH)], nib, nsi)

        @pl.when(jnp.logical_not(first))
        def _():
            pltpu.make_async_copy(wb, acc_sh.at[dstsb], semw).wait()
            pltpu.make_async_copy(dnb, dn_sh.at[dstdb], semd).wait()

        for g in range(_CH // 16):
            v = ib[pl.ds(2 * _CH + 16 * g, 16)]
            dstsb[pl.ds(16 * g, 16)] = v
            dstdb[pl.ds(16 * g, 16)] = lax.shift_right_logical(v, 3)

        @pl.when(k + 1 < kc)
        def _():
            pltpu.make_async_copy(
                idxp_hbm.at[pl.ds(0, 4 * _CH)], nib, nsi).wait()
            pltpu.async_copy(xl_hbm.at[idx_slice(nib, 0)], nxb, nsl)
            pltpu.async_copy(xr_hbm.at[idx_slice(nib, 1)], nrb, nsr)

        pltpu.make_async_copy(xl_hbm.at[idx_slice(ib, 0)], xb, ssl).wait()
        pltpu.make_async_copy(xr_hbm.at[idx_slice(ib, 1)], rb, ssr).wait()

        def edge_body(e, carry_):
            efull = jnp.full((16,), e, jnp.int32)
            a_splat = plsc.load_gather(ib, [efull + 3 * _CH])
            j_splat = jnp.bitwise_and(
                plsc.load_gather(ib, [efull + 2 * _CH]), 7)
            rowb = a_splat * 128 + col
            dnv = jnp.zeros((16,), jnp.float32)
            for h in range(8):
                xlv = xb[e, pl.ds(16 * h, 16)]
                xrv = rb[e, pl.ds(16 * h, 16)]
                eev = plsc.load_gather(tab, [rowb + 16 * h])
                s = xlv + xrv + eev
                z = jnp.where(s >= 0, s, 0.2 * s)
                alpha = jnp.sum(z * att_regs[h])
                exv = jnp.exp(jnp.full((16,), alpha))
                wb[e, pl.ds(16 * h, 16)] = xlv * exv
                dnv = jnp.where(col == h, exv, dnv)
            zero = jnp.zeros((16,), jnp.float32)
            for j in range(8):
                dnb[e, pl.ds(16 * j, 16)] = jnp.where(j_splat == j, dnv, zero)
            return carry_

        lax.fori_loop(0, _CH, edge_body, 0)
        pltpu.async_copy(wb, acc_sh.at[dstsb], semw, add=True)
        pltpu.async_copy(dnb, dn_sh.at[dstdb], semd, add=True)

    def pair_body(k2, carry):
        k = 2 * k2
        process(k, k2 == 0, idxb0, xlb0, xrb0, sl0, sr0,
                idxb1, xlb1, xrb1, si1, sl1, sr1)
        process(k + 1, jnp.bool_(False), idxb1, xlb1, xrb1, sl1, sr1,
                idxb0, xlb0, xrb0, si0, sl0, sr0)
        return carry

    lax.fori_loop(0, kc // 2, pair_body, 0)
    pltpu.make_async_copy(wb, acc_sh.at[dstsb], semw).wait()
    pltpu.make_async_copy(dnb, dn_sh.at[dstdb], semd).wait()
    plsc.subcore_barrier()
    for off, sz in _row_chunks(rows):
        pltpu.sync_copy(acc_sh.at[pl.ds(r0 + off, sz)], wb.at[pl.ds(0, sz)])
        pltpu.sync_copy(wb.at[pl.ds(0, sz)],
                        accp_hbm.at[cid, pl.ds(r0 + off, sz)])
    for off, sz in _row_chunks(rows_d):
        pltpu.sync_copy(dn_sh.at[pl.ds(r0d + off, sz)], wb.at[pl.ds(0, sz)])
        pltpu.sync_copy(wb.at[pl.ds(0, sz)],
                        dnp_hbm.at[cid, pl.ds(r0d + off, sz)])


def kernel(x, edge_index, edge_attr, node_types, nt_emb, et_emb,
           Wl0, bl0, Wr0, br0, We0, att0, bias0,
           Wl1, bl1, Wr1, br1, We1, att1, bias1,
           Wo, bo):
    n, d = x.shape
    e_num = edge_attr.shape[0]
    f32 = jnp.float32

    kc = -(-e_num // (_NW * _CH))
    kc += kc % 2
    e_pad = _NW * _CH * kc
    pad = e_pad - e_num
    idt = edge_index.dtype
    src = jnp.concatenate([edge_index[0], jnp.zeros((pad,), idt)])
    dstg = jnp.concatenate([edge_index[1], jnp.zeros((pad,), idt)])
    dsts = jnp.concatenate([edge_index[1], jnp.full((pad,), n, idt)])
    attr = jnp.concatenate([edge_attr.astype(idt), jnp.zeros((pad,), idt)])
    idxp = (jnp.stack([src, dstg, dsts, attr])
            .reshape(4, _NW * kc, _CH)
            .transpose(1, 0, 2)
            .reshape(-1))
    n_acc = -(-(n + 1) // 128) * 128
    n_dn = -(-n_acc // 1024) * 128

    jidx = jnp.arange(128) // 16
    seg = (jidx[:, None] == jidx[None, :]).astype(f32)
    x16 = (jnp.arange(16)[:, None] == jidx[None, :]).astype(f32)

    r = 1000
    gn = n // r

    h0 = pl.pallas_call(
        _h0_body,
        grid=(gn,),
        in_specs=[pl.BlockSpec((r, 1), lambda i: (i, 0)),
                  pl.BlockSpec((r, 128), lambda i: (i, 0)),
                  pl.BlockSpec((5, 128), lambda i: (0, 0))],
        out_specs=pl.BlockSpec((r, 128), lambda i: (i, 0)),
        out_shape=jax.ShapeDtypeStruct((n, 128), f32),
    )(node_types.reshape(n, 1), x, nt_emb)

    eb = e_num // 128
    ebs = eb
    nsteps = 1
    tab8 = pl.pallas_call(
        functools.partial(_counts_body, nsteps=nsteps, e_total=float(e_num)),
        grid=(nsteps,),
        in_specs=[pl.BlockSpec((ebs, 128), lambda i: (i, 0)),
                  pl.BlockSpec((3, 128), lambda i: (0, 0))],
        out_specs=pl.BlockSpec((8, 128), lambda i: (0, 0)),
        out_shape=jax.ShapeDtypeStruct((8, 128), f32),
        scratch_shapes=[pltpu.SMEM((3,), f32)],
    )(edge_attr.reshape(eb, 128), et_emb)

    proj_call = pl.pallas_call(
        _proj_body,
        grid=(gn,),
        in_specs=[pl.BlockSpec((r, 128), lambda i: (i, 0)),
                  pl.BlockSpec((128, 128), lambda i: (0, 0)),
                  pl.BlockSpec((1, 128), lambda i: (0, 0)),
                  pl.BlockSpec((128, 128), lambda i: (0, 0)),
                  pl.BlockSpec((1, 128), lambda i: (0, 0))],
        out_specs=[pl.BlockSpec((r, 128), lambda i: (i, 0)),
                   pl.BlockSpec((r, 128), lambda i: (i, 0))],
        out_shape=[jax.ShapeDtypeStruct((n, 128), f32),
                   jax.ShapeDtypeStruct((n, 128), f32)],
    )

    tabproj_call = pl.pallas_call(
        _tabproj_body,
        out_shape=jax.ShapeDtypeStruct((8, 128), f32),
    )

    def epi_call(final, nout):
        extra = ([pl.BlockSpec((128, 128), lambda i: (0, 0)),
                  pl.BlockSpec((1, 128), lambda i: (0, 0))] if final else [])
        return pl.pallas_call(
            functools.partial(_epi_body, final=final),
            grid=(gn,),
            in_specs=[pl.BlockSpec((r, 128), lambda i: (i, 0)),
                      pl.BlockSpec((r, 128), lambda i: (i, 0)),
                      pl.BlockSpec((r, 128), lambda i: (i, 0)),
                      pl.BlockSpec((2, r, 128), lambda i: (0, i, 0)),
                      pl.BlockSpec((2, r, 16), lambda i: (0, i, 0)),
                      pl.BlockSpec((8, 128), lambda i: (0, 0)),
                      pl.BlockSpec((1, 128), lambda i: (0, 0)),
                      pl.BlockSpec((128, 128), lambda i: (0, 0)),
                      pl.BlockSpec((16, 128), lambda i: (0, 0)),
                      pl.BlockSpec((1, 128), lambda i: (0, 0))] + extra,
            out_specs=pl.BlockSpec((r, 128), lambda i: (i, 0)),
            out_shape=jax.ShapeDtypeStruct((nout, 128), f32),
        )

    h = h0
    layers = [(Wl0, bl0, Wr0, br0, We0, att0, bias0),
              (Wl1, bl1, Wr1, br1, We1, att1, bias1)]
    for li, (wl, bl, wr, br, we, att, bias) in enumerate(layers):
        proj8 = tabproj_call(tab8, we)
        xl, xr = proj_call(h, wl, bl.reshape(1, 128), wr, br.reshape(1, 128))
        accp, dnp = _sc_edge_pass(idxp, xl, xr, proj8, att,
                                  kc=kc, n_acc=n_acc, n_dn=n_dn)
        final = li == 1
        args = [h, xl, xr, accp, dnp, proj8, att.reshape(1, 128), seg, x16,
                bias.reshape(1, 128)]
        if final:
            args += [Wo, bo.reshape(1, 128)]
        h = epi_call(final, n)(*args)
    return h


def _sc_edge_pass(idxp, xl, xr, proj8, att, *, kc, n_acc, n_dn):
    f32 = jnp.float32
    mesh = plsc.VectorSubcoreMesh(core_axis_name="c", subcore_axis_name="s",
                                  num_cores=_NC, num_subcores=_NS)
    sc_call = pl.kernel(
        functools.partial(_sc_body, kc=kc, n_acc=n_acc, n_dn=n_dn),
        out_type=(jax.ShapeDtypeStruct((_NC, n_acc, 128), f32),
                  jax.ShapeDtypeStruct((_NC, n_dn, 128), f32)),
        mesh=mesh,
        compiler_params=pltpu.CompilerParams(needs_layout_passes=False),
        scratch_types=[
            pltpu.VMEM((4 * _CH,), jnp.int32),
            pltpu.VMEM((4 * _CH,), jnp.int32),
            pltpu.VMEM((_CH,), jnp.int32),
            pltpu.VMEM((_CH,), jnp.int32),
            pltpu.VMEM((_CH, 128), f32),
            pltpu.VMEM((_CH, 128), f32),
            pltpu.VMEM((_CH, 128), f32),
            pltpu.VMEM((_CH, 128), f32),
            pltpu.VMEM((_CH, 128), f32),
            pltpu.VMEM((_CH, 128), f32),
            pltpu.VMEM((1024,), f32),
            pltpu.VMEM((1, 128), f32),
            pltpu.MemorySpace.VMEM_SHARED((n_acc, 128), f32),
            pltpu.MemorySpace.VMEM_SHARED((n_dn, 128), f32),
        ] + [pltpu.SemaphoreType.DMA] * 8,
    )
    accp, dnp = sc_call(idxp, xl, xr, proj8.reshape(-1), att.reshape(1, 128))
    return accp, dnp.reshape(_NC, n_dn * 8, 16)


# speedup vs baseline: 20.0249x; 1.0059x over previous
import functools

import jax
import jax.numpy as jnp
from jax import lax
from jax.experimental import pallas as pl
from jax.experimental.pallas import tpu as pltpu
from jax.experimental.pallas import tpu_sc as plsc

_NC = 2
_NS = 16
_NW = _NC * _NS
_CH = 48


def _h0_body(t_ref, x_ref, nt_ref, h_ref):
    t = t_ref[...]
    h = x_ref[...] + jnp.broadcast_to(nt_ref[0:1, :], x_ref.shape)
    for k in range(1, 5):
        h = jnp.where(t == k, x_ref[...] + nt_ref[k:k + 1, :], h)
    h_ref[...] = h


def _counts_body(a_ref, et_ref, tab_ref, cnt_ref, *, nsteps, e_total):
    i = pl.program_id(0)

    @pl.when(i == 0)
    def _():
        for c in range(3):
            cnt_ref[c] = 0.0

    blk = a_ref[...]
    for c in range(3):
        cnt_ref[c] += jnp.sum((blk == c).astype(jnp.float32))

    @pl.when(i == nsteps - 1)
    def _():
        inv = 1.0 / e_total
        mean = (cnt_ref[0] * et_ref[0:1, :] + cnt_ref[1] * et_ref[1:2, :]
                + cnt_ref[2] * et_ref[2:3, :]) * inv
        tab_ref[...] = jnp.concatenate(
            [et_ref[...], mean, jnp.zeros((4, 128), jnp.float32)], axis=0)


def _tabproj_body(tab_ref, we_ref, out_ref):
    out_ref[...] = lax.dot_general(
        tab_ref[...], we_ref[...], (((1,), (1,)), ((), ())),
        preferred_element_type=jnp.float32)


def _proj_body(h_ref, wl_ref, bl_ref, wr_ref, br_ref, xl_ref, xr_ref):
    h = h_ref[...]
    xl_ref[...] = lax.dot_general(
        h, wl_ref[...], (((1,), (1,)), ((), ())),
        preferred_element_type=jnp.float32) + bl_ref[...]
    xr_ref[...] = lax.dot_general(
        h, wr_ref[...], (((1,), (1,)), ((), ())),
        preferred_element_type=jnp.float32) + br_ref[...]


def _epi_body(h_ref, xl_ref, xr_ref, acc_ref, dn_ref, proj_ref, attf_ref,
              seg_ref, x16_ref, bias_ref, *rest, final):
    if final:
        wo_ref, bo_ref, out_ref = rest
    else:
        (out_ref,) = rest
    xl = xl_ref[...]
    acc = acc_ref[0] + acc_ref[1]
    dn16 = dn_ref[0] + dn_ref[1]
    s = xl + xr_ref[...] + proj_ref[3:4, :]
    z = jnp.where(s >= 0, s, 0.2 * s)
    alpha128 = lax.dot_general(
        z * attf_ref[...], seg_ref[...], (((1,), (0,)), ((), ())),
        preferred_element_type=jnp.float32)
    ex128 = jnp.exp(alpha128)
    dn128 = lax.dot_general(
        dn16, x16_ref[...], (((1,), (0,)), ((), ())),
        preferred_element_type=jnp.float32)
    g = (acc + xl * ex128) / (dn128 + ex128)
    hn = jnp.maximum(g + bias_ref[...], 0.0) + h_ref[...]
    if final:
        out_ref[...] = lax.dot_general(
            hn, wo_ref[...], (((1,), (1,)), ((), ())),
            preferred_element_type=jnp.float32) + bo_ref[...]
    else:
        out_ref[...] = hn


def _row_chunks(rows):
    nfull, rem = divmod(rows, _CH)
    sizes = [_CH] * nfull + ([rem] if rem else [])
    offs, o = [], 0
    for s in sizes:
        offs.append(o)
        o += s
    return list(zip(offs, sizes))


def _sc_body(idxp_hbm, xl_hbm, xr_hbm, tab_hbm,
             att_hbm, accp_hbm, dnp_hbm,
             idxb0, idxb1, dstsb, dstdb, xlb0, xlb1, xrb0, xrb1, wb, dnb,
             tab, attv, acc_sh, dn_sh,
             si0, si1, sl0, sl1, sr0, sr1, semw, semd, *, kc, n_acc, n_dn):
    cid = lax.axis_index("c")
    sid = lax.axis_index("s")
    wid = sid * _NC + cid
    rows = n_acc // _NS
    rows_d = n_dn // _NS
    r0 = sid * rows
    r0d = sid * rows_d

    def zrow_body(i, carry):
        for kk in range(8):
            wb[i, pl.ds(16 * kk, 16)] = jnp.zeros((16,), jnp.float32)
        return carry

    lax.fori_loop(0, _CH, zrow_body, 0)
    for off, sz in _row_chunks(rows):
        pltpu.sync_copy(wb.at[pl.ds(0, sz)], acc_sh.at[pl.ds(r0 + off, sz)])
    for off, sz in _row_chunks(rows_d):
        pltpu.sync_copy(wb.at[pl.ds(0, sz)], dn_sh.at[pl.ds(r0d + off, sz)])
    pltpu.sync_copy(tab_hbm, tab)
    pltpu.sync_copy(att_hbm, attv)
    plsc.subcore_barrier()

    col = lax.iota(jnp.int32, 16)
    att_regs = [attv[0, pl.ds(16 * h, 16)] for h in range(8)]
    base0 = wid * kc

    def idx_slice(ib, part):
        return ib.at[pl.ds(part * _CH, _CH)]

    pltpu.sync_copy(idxp_hbm.at[pl.ds(base0 * 4 * _CH, 4 * _CH)], idxb0)
    pltpu.async_copy(xl_hbm.at[idx_slice(idxb0, 0)], xlb0, sl0)
    pltpu.async_copy(xr_hbm.at[idx_slice(idxb0, 1)], xrb0, sr0)

    def process(k, first, ib, xb, rb, ssl, ssr, nib, nxb, nrb, nsi, nsl, nsr):
        @pl.when(k + 1 < kc)
        def _():
            nbase = (base0 + k + 1) * (4 * _CH)
            pltpu.async_copy(idxp_hbm.at[pl.ds(nbase, 4 * _CH)], nib, nsi)

        @pl.when(jnp.logical_not(first))
        def _():
            pltpu.make_async_copy(wb, acc_sh.at[dstsb], semw).wait()
            pltpu.make_async_copy(dnb, dn_sh.at[dstdb], semd).wait()

        for g in range(_CH // 16):
            v = ib[pl.ds(2 * _CH + 16 * g, 16)]
            dstsb[pl.ds(16 * g, 16)] = v
            dstdb[pl.ds(16 * g, 16)] = lax.shift_right_logical(v, 3)

        @pl.when(k + 1 < kc)
        def _():
            pltpu.make_async_copy(
                idxp_hbm.at[pl.ds(0, 4 * _CH)], nib, nsi).wait()
            pltpu.async_copy(xl_hbm.at[idx_slice(nib, 0)], nxb, nsl)
            pltpu.async_copy(xr_hbm.at[idx_slice(nib, 1)], nrb, nsr)

        pltpu.make_async_copy(xl_hbm.at[idx_slice(ib, 0)], xb, ssl).wait()
        pltpu.make_async_copy(xr_hbm.at[idx_slice(ib, 1)], rb, ssr).wait()

        def do_edge(e):
            efull = jnp.full((16,), e, jnp.int32)
            a_splat = plsc.load_gather(ib, [efull + 3 * _CH])
            j_splat = jnp.bitwise_and(
                plsc.load_gather(ib, [efull + 2 * _CH]), 7)
            rowb = a_splat * 128 + col
            dnv = jnp.zeros((16,), jnp.float32)
            for h in range(8):
                xlv = xb[e, pl.ds(16 * h, 16)]
                xrv = rb[e, pl.ds(16 * h, 16)]
                eev = plsc.load_gather(tab, [rowb + 16 * h])
                s = xlv + xrv + eev
                z = jnp.where(s >= 0, s, 0.2 * s)
                alpha = jnp.sum(z * att_regs[h])
                exv = jnp.exp(jnp.full((16,), alpha))
                wb[e, pl.ds(16 * h, 16)] = xlv * exv
                dnv = jnp.where(col == h, exv, dnv)
            zero = jnp.zeros((16,), jnp.float32)
            for j in range(8):
                dnb[e, pl.ds(16 * j, 16)] = jnp.where(j_splat == j, dnv, zero)

        def edge_body(e2, carry_):
            do_edge(2 * e2)
            do_edge(2 * e2 + 1)
            return carry_

        lax.fori_loop(0, _CH // 2, edge_body, 0)
        pltpu.async_copy(wb, acc_sh.at[dstsb], semw, add=True)
        pltpu.async_copy(dnb, dn_sh.at[dstdb], semd, add=True)

    def pair_body(k2, carry):
        k = 2 * k2
        process(k, k2 == 0, idxb0, xlb0, xrb0, sl0, sr0,
                idxb1, xlb1, xrb1, si1, sl1, sr1)
        process(k + 1, jnp.bool_(False), idxb1, xlb1, xrb1, sl1, sr1,
                idxb0, xlb0, xrb0, si0, sl0, sr0)
        return carry

    lax.fori_loop(0, kc // 2, pair_body, 0)
    pltpu.make_async_copy(wb, acc_sh.at[dstsb], semw).wait()
    pltpu.make_async_copy(dnb, dn_sh.at[dstdb], semd).wait()
    plsc.subcore_barrier()
    for off, sz in _row_chunks(rows):
        pltpu.sync_copy(acc_sh.at[pl.ds(r0 + off, sz)], wb.at[pl.ds(0, sz)])
        pltpu.sync_copy(wb.at[pl.ds(0, sz)],
                        accp_hbm.at[cid, pl.ds(r0 + off, sz)])
    for off, sz in _row_chunks(rows_d):
        pltpu.sync_copy(dn_sh.at[pl.ds(r0d + off, sz)], wb.at[pl.ds(0, sz)])
        pltpu.sync_copy(wb.at[pl.ds(0, sz)],
                        dnp_hbm.at[cid, pl.ds(r0d + off, sz)])


def kernel(x, edge_index, edge_attr, node_types, nt_emb, et_emb,
           Wl0, bl0, Wr0, br0, We0, att0, bias0,
           Wl1, bl1, Wr1, br1, We1, att1, bias1,
           Wo, bo):
    n, d = x.shape
    e_num = edge_attr.shape[0]
    f32 = jnp.float32

    kc = -(-e_num // (_NW * _CH))
    kc += kc % 2
    e_pad = _NW * _CH * kc
    pad = e_pad - e_num
    idt = edge_index.dtype
    src = jnp.concatenate([edge_index[0], jnp.zeros((pad,), idt)])
    dstg = jnp.concatenate([edge_index[1], jnp.zeros((pad,), idt)])
    dsts = jnp.concatenate([edge_index[1], jnp.full((pad,), n, idt)])
    attr = jnp.concatenate([edge_attr.astype(idt), jnp.zeros((pad,), idt)])
    idxp = (jnp.stack([src, dstg, dsts, attr])
            .reshape(4, _NW * kc, _CH)
            .transpose(1, 0, 2)
            .reshape(-1))
    n_acc = -(-(n + 1) // 128) * 128
    n_dn = -(-n_acc // 1024) * 128

    jidx = jnp.arange(128) // 16
    seg = (jidx[:, None] == jidx[None, :]).astype(f32)
    x16 = (jnp.arange(16)[:, None] == jidx[None, :]).astype(f32)

    r = 1000
    gn = n // r

    h0 = pl.pallas_call(
        _h0_body,
        grid=(gn,),
        in_specs=[pl.BlockSpec((r, 1), lambda i: (i, 0)),
                  pl.BlockSpec((r, 128), lambda i: (i, 0)),
                  pl.BlockSpec((5, 128), lambda i: (0, 0))],
        out_specs=pl.BlockSpec((r, 128), lambda i: (i, 0)),
        out_shape=jax.ShapeDtypeStruct((n, 128), f32),
    )(node_types.reshape(n, 1), x, nt_emb)

    eb = e_num // 128
    ebs = eb
    nsteps = 1
    tab8 = pl.pallas_call(
        functools.partial(_counts_body, nsteps=nsteps, e_total=float(e_num)),
        grid=(nsteps,),
        in_specs=[pl.BlockSpec((ebs, 128), lambda i: (i, 0)),
                  pl.BlockSpec((3, 128), lambda i: (0, 0))],
        out_specs=pl.BlockSpec((8, 128), lambda i: (0, 0)),
        out_shape=jax.ShapeDtypeStruct((8, 128), f32),
        scratch_shapes=[pltpu.SMEM((3,), f32)],
    )(edge_attr.reshape(eb, 128), et_emb)

    proj_call = pl.pallas_call(
        _proj_body,
        grid=(gn,),
        in_specs=[pl.BlockSpec((r, 128), lambda i: (i, 0)),
                  pl.BlockSpec((128, 128), lambda i: (0, 0)),
                  pl.BlockSpec((1, 128), lambda i: (0, 0)),
                  pl.BlockSpec((128, 128), lambda i: (0, 0)),
                  pl.BlockSpec((1, 128), lambda i: (0, 0))],
        out_specs=[pl.BlockSpec((r, 128), lambda i: (i, 0)),
                   pl.BlockSpec((r, 128), lambda i: (i, 0))],
        out_shape=[jax.ShapeDtypeStruct((n, 128), f32),
                   jax.ShapeDtypeStruct((n, 128), f32)],
    )

    tabproj_call = pl.pallas_call(
        _tabproj_body,
        out_shape=jax.ShapeDtypeStruct((8, 128), f32),
    )

    def epi_call(final, nout):
        extra = ([pl.BlockSpec((128, 128), lambda i: (0, 0)),
                  pl.BlockSpec((1, 128), lambda i: (0, 0))] if final else [])
        return pl.pallas_call(
            functools.partial(_epi_body, final=final),
            grid=(gn,),
            in_specs=[pl.BlockSpec((r, 128), lambda i: (i, 0)),
                      pl.BlockSpec((r, 128), lambda i: (i, 0)),
                      pl.BlockSpec((r, 128), lambda i: (i, 0)),
                      pl.BlockSpec((2, r, 128), lambda i: (0, i, 0)),
                      pl.BlockSpec((2, r, 16), lambda i: (0, i, 0)),
                      pl.BlockSpec((8, 128), lambda i: (0, 0)),
                      pl.BlockSpec((1, 128), lambda i: (0, 0)),
                      pl.BlockSpec((128, 128), lambda i: (0, 0)),
                      pl.BlockSpec((16, 128), lambda i: (0, 0)),
                      pl.BlockSpec((1, 128), lambda i: (0, 0))] + extra,
            out_specs=pl.BlockSpec((r, 128), lambda i: (i, 0)),
            out_shape=jax.ShapeDtypeStruct((nout, 128), f32),
        )

    h = h0
    layers = [(Wl0, bl0, Wr0, br0, We0, att0, bias0),
              (Wl1, bl1, Wr1, br1, We1, att1, bias1)]
    for li, (wl, bl, wr, br, we, att, bias) in enumerate(layers):
        proj8 = tabproj_call(tab8, we)
        xl, xr = proj_call(h, wl, bl.reshape(1, 128), wr, br.reshape(1, 128))
        accp, dnp = _sc_edge_pass(idxp, xl, xr, proj8, att,
                                  kc=kc, n_acc=n_acc, n_dn=n_dn)
        final = li == 1
        args = [h, xl, xr, accp, dnp, proj8, att.reshape(1, 128), seg, x16,
                bias.reshape(1, 128)]
        if final:
            args += [Wo, bo.reshape(1, 128)]
        h = epi_call(final, n)(*args)
    return h


def _sc_edge_pass(idxp, xl, xr, proj8, att, *, kc, n_acc, n_dn):
    f32 = jnp.float32
    mesh = plsc.VectorSubcoreMesh(core_axis_name="c", subcore_axis_name="s",
                                  num_cores=_NC, num_subcores=_NS)
    sc_call = pl.kernel(
        functools.partial(_sc_body, kc=kc, n_acc=n_acc, n_dn=n_dn),
        out_type=(jax.ShapeDtypeStruct((_NC, n_acc, 128), f32),
                  jax.ShapeDtypeStruct((_NC, n_dn, 128), f32)),
        mesh=mesh,
        compiler_params=pltpu.CompilerParams(needs_layout_passes=False),
        scratch_types=[
            pltpu.VMEM((4 * _CH,), jnp.int32),
            pltpu.VMEM((4 * _CH,), jnp.int32),
            pltpu.VMEM((_CH,), jnp.int32),
            pltpu.VMEM((_CH,), jnp.int32),
            pltpu.VMEM((_CH, 128), f32),
            pltpu.VMEM((_CH, 128), f32),
            pltpu.VMEM((_CH, 128), f32),
            pltpu.VMEM((_CH, 128), f32),
            pltpu.VMEM((_CH, 128), f32),
            pltpu.VMEM((_CH, 128), f32),
            pltpu.VMEM((1024,), f32),
            pltpu.VMEM((1, 128), f32),
            pltpu.MemorySpace.VMEM_SHARED((n_acc, 128), f32),
            pltpu.MemorySpace.VMEM_SHARED((n_dn, 128), f32),
        ] + [pltpu.SemaphoreType.DMA] * 8,
    )
    accp, dnp = sc_call(idxp, xl, xr, proj8.reshape(-1), att.reshape(1, 128))
    return accp, dnp.reshape(_NC, n_dn * 8, 16)


# speedup vs baseline: 20.0451x; 1.0010x over previous
import functools

import jax
import jax.numpy as jnp
from jax import lax
from jax.experimental import pallas as pl
from jax.experimental.pallas import tpu as pltpu
from jax.experimental.pallas import tpu_sc as plsc

_NC = 2
_NS = 16
_NW = _NC * _NS
_CH = 48


def _h0_body(t_ref, x_ref, nt_ref, h_ref):
    t = t_ref[...]
    h = x_ref[...] + jnp.broadcast_to(nt_ref[0:1, :], x_ref.shape)
    for k in range(1, 5):
        h = jnp.where(t == k, x_ref[...] + nt_ref[k:k + 1, :], h)
    h_ref[...] = h


def _counts_body(a_ref, et_ref, tab_ref, cnt_ref, *, nsteps, e_total):
    i = pl.program_id(0)

    @pl.when(i == 0)
    def _():
        for c in range(3):
            cnt_ref[c] = 0.0

    blk = a_ref[...]
    for c in range(3):
        cnt_ref[c] += jnp.sum((blk == c).astype(jnp.float32))

    @pl.when(i == nsteps - 1)
    def _():
        inv = 1.0 / e_total
        mean = (cnt_ref[0] * et_ref[0:1, :] + cnt_ref[1] * et_ref[1:2, :]
                + cnt_ref[2] * et_ref[2:3, :]) * inv
        tab_ref[...] = jnp.concatenate(
            [et_ref[...], mean, jnp.zeros((4, 128), jnp.float32)], axis=0)


def _tabproj_body(tab_ref, we_ref, out_ref):
    out_ref[...] = lax.dot_general(
        tab_ref[...], we_ref[...], (((1,), (1,)), ((), ())),
        preferred_element_type=jnp.float32)


def _proj_body(h_ref, wl_ref, bl_ref, wr_ref, br_ref, xl_ref, xr_ref):
    h = h_ref[...]
    xl_ref[...] = lax.dot_general(
        h, wl_ref[...], (((1,), (1,)), ((), ())),
        preferred_element_type=jnp.float32) + bl_ref[...]
    xr_ref[...] = lax.dot_general(
        h, wr_ref[...], (((1,), (1,)), ((), ())),
        preferred_element_type=jnp.float32) + br_ref[...]


def _epi_body(h_ref, xl_ref, xr_ref, acc_ref, dn_ref, proj_ref, attf_ref,
              seg_ref, x16_ref, bias_ref, *rest, final):
    if final:
        wo_ref, bo_ref, out_ref = rest
    else:
        (out_ref,) = rest
    xl = xl_ref[...]
    acc = acc_ref[0] + acc_ref[1]
    dn16 = dn_ref[0] + dn_ref[1]
    s = xl + xr_ref[...] + proj_ref[3:4, :]
    z = jnp.where(s >= 0, s, 0.2 * s)
    alpha128 = lax.dot_general(
        z * attf_ref[...], seg_ref[...], (((1,), (0,)), ((), ())),
        preferred_element_type=jnp.float32)
    ex128 = jnp.exp(alpha128)
    dn128 = lax.dot_general(
        dn16, x16_ref[...], (((1,), (0,)), ((), ())),
        preferred_element_type=jnp.float32)
    g = (acc + xl * ex128) / (dn128 + ex128)
    hn = jnp.maximum(g + bias_ref[...], 0.0) + h_ref[...]
    if final:
        out_ref[...] = lax.dot_general(
            hn, wo_ref[...], (((1,), (1,)), ((), ())),
            preferred_element_type=jnp.float32) + bo_ref[...]
    else:
        out_ref[...] = hn


def _row_chunks(rows):
    nfull, rem = divmod(rows, _CH)
    sizes = [_CH] * nfull + ([rem] if rem else [])
    offs, o = [], 0
    for s in sizes:
        offs.append(o)
        o += s
    return list(zip(offs, sizes))


def _sc_body(idxp_hbm, xl_hbm, xr_hbm, tab_hbm,
             att_hbm, accp_hbm, dnp_hbm,
             idxb0, idxb1, dstsb0, dstdb0, dstsb1, dstdb1,
             xlb0, xlb1, xrb0, xrb1, wb, dnb,
             tab, attv, acc_sh, dn_sh,
             si0, si1, sl0, sl1, sr0, sr1, semw, semd, *, kc, n_acc, n_dn):
    cid = lax.axis_index("c")
    sid = lax.axis_index("s")
    wid = sid * _NC + cid
    rows = n_acc // _NS
    rows_d = n_dn // _NS
    r0 = sid * rows
    r0d = sid * rows_d

    def zrow_body(i, carry):
        for kk in range(8):
            wb[i, pl.ds(16 * kk, 16)] = jnp.zeros((16,), jnp.float32)
        return carry

    lax.fori_loop(0, _CH, zrow_body, 0)
    for off, sz in _row_chunks(rows):
        pltpu.sync_copy(wb.at[pl.ds(0, sz)], acc_sh.at[pl.ds(r0 + off, sz)])
    for off, sz in _row_chunks(rows_d):
        pltpu.sync_copy(wb.at[pl.ds(0, sz)], dn_sh.at[pl.ds(r0d + off, sz)])
    pltpu.sync_copy(tab_hbm, tab)
    pltpu.sync_copy(att_hbm, attv)
    plsc.subcore_barrier()

    col = lax.iota(jnp.int32, 16)
    att_regs = [attv[0, pl.ds(16 * h, 16)] for h in range(8)]
    base0 = wid * kc

    def idx_slice(ib, part):
        return ib.at[pl.ds(part * _CH, _CH)]

    pltpu.sync_copy(idxp_hbm.at[pl.ds(base0 * 4 * _CH, 4 * _CH)], idxb0)
    pltpu.async_copy(xl_hbm.at[idx_slice(idxb0, 0)], xlb0, sl0)
    pltpu.async_copy(xr_hbm.at[idx_slice(idxb0, 1)], xrb0, sr0)

    def process(k, first, ib, xb, rb, dsb, ddb, ssl, ssr,
                nib, nxb, nrb, nsi, nsl, nsr):
        @pl.when(k + 1 < kc)
        def _():
            nbase = (base0 + k + 1) * (4 * _CH)
            pltpu.async_copy(idxp_hbm.at[pl.ds(nbase, 4 * _CH)], nib, nsi)

        for g in range(_CH // 16):
            v = ib[pl.ds(2 * _CH + 16 * g, 16)]
            dsb[pl.ds(16 * g, 16)] = v
            ddb[pl.ds(16 * g, 16)] = lax.shift_right_logical(v, 3)

        @pl.when(k + 1 < kc)
        def _():
            pltpu.make_async_copy(
                idxp_hbm.at[pl.ds(0, 4 * _CH)], nib, nsi).wait()
            pltpu.async_copy(xl_hbm.at[idx_slice(nib, 0)], nxb, nsl)
            pltpu.async_copy(xr_hbm.at[idx_slice(nib, 1)], nrb, nsr)

        pltpu.make_async_copy(xl_hbm.at[idx_slice(ib, 0)], xb, ssl).wait()
        pltpu.make_async_copy(xr_hbm.at[idx_slice(ib, 1)], rb, ssr).wait()

        @pl.when(jnp.logical_not(first))
        def _():
            pltpu.make_async_copy(wb, acc_sh.at[dsb], semw).wait()
            pltpu.make_async_copy(dnb, dn_sh.at[ddb], semd).wait()

        def do_edge(e):
            efull = jnp.full((16,), e, jnp.int32)
            a_splat = plsc.load_gather(ib, [efull + 3 * _CH])
            j_splat = jnp.bitwise_and(
                plsc.load_gather(ib, [efull + 2 * _CH]), 7)
            rowb = a_splat * 128 + col
            dnv = jnp.zeros((16,), jnp.float32)
            for h in range(8):
                xlv = xb[e, pl.ds(16 * h, 16)]
                xrv = rb[e, pl.ds(16 * h, 16)]
                eev = plsc.load_gather(tab, [rowb + 16 * h])
                s = xlv + xrv + eev
                z = jnp.where(s >= 0, s, 0.2 * s)
                alpha = jnp.sum(z * att_regs[h])
                exv = jnp.exp(jnp.full((16,), alpha))
                wb[e, pl.ds(16 * h, 16)] = xlv * exv
                dnv = jnp.where(col == h, exv, dnv)
            zero = jnp.zeros((16,), jnp.float32)
            for j in range(8):
                dnb[e, pl.ds(16 * j, 16)] = jnp.where(j_splat == j, dnv, zero)

        def edge_body(e2, carry_):
            do_edge(2 * e2)
            do_edge(2 * e2 + 1)
            return carry_

        lax.fori_loop(0, _CH // 2, edge_body, 0)
        pltpu.async_copy(wb, acc_sh.at[dsb], semw, add=True)
        pltpu.async_copy(dnb, dn_sh.at[ddb], semd, add=True)

    def pair_body(k2, carry):
        k = 2 * k2
        process(k, k2 == 0, idxb0, xlb0, xrb0, dstsb0, dstdb0, sl0, sr0,
                idxb1, xlb1, xrb1, si1, sl1, sr1)
        process(k + 1, jnp.bool_(False), idxb1, xlb1, xrb1, dstsb1, dstdb1,
                sl1, sr1, idxb0, xlb0, xrb0, si0, sl0, sr0)
        return carry

    lax.fori_loop(0, kc // 2, pair_body, 0)
    pltpu.make_async_copy(wb, acc_sh.at[dstsb1], semw).wait()
    pltpu.make_async_copy(dnb, dn_sh.at[dstdb1], semd).wait()
    plsc.subcore_barrier()
    for off, sz in _row_chunks(rows):
        pltpu.sync_copy(acc_sh.at[pl.ds(r0 + off, sz)], wb.at[pl.ds(0, sz)])
        pltpu.sync_copy(wb.at[pl.ds(0, sz)],
                        accp_hbm.at[cid, pl.ds(r0 + off, sz)])
    for off, sz in _row_chunks(rows_d):
        pltpu.sync_copy(dn_sh.at[pl.ds(r0d + off, sz)], wb.at[pl.ds(0, sz)])
        pltpu.sync_copy(wb.at[pl.ds(0, sz)],
                        dnp_hbm.at[cid, pl.ds(r0d + off, sz)])


def kernel(x, edge_index, edge_attr, node_types, nt_emb, et_emb,
           Wl0, bl0, Wr0, br0, We0, att0, bias0,
           Wl1, bl1, Wr1, br1, We1, att1, bias1,
           Wo, bo):
    n, d = x.shape
    e_num = edge_attr.shape[0]
    f32 = jnp.float32

    kc = -(-e_num // (_NW * _CH))
    kc += kc % 2
    e_pad = _NW * _CH * kc
    pad = e_pad - e_num
    idt = edge_index.dtype
    src = jnp.concatenate([edge_index[0], jnp.zeros((pad,), idt)])
    dstg = jnp.concatenate([edge_index[1], jnp.zeros((pad,), idt)])
    dsts = jnp.concatenate([edge_index[1], jnp.full((pad,), n, idt)])
    attr = jnp.concatenate([edge_attr.astype(idt), jnp.zeros((pad,), idt)])
    idxp = (jnp.stack([src, dstg, dsts, attr])
            .reshape(4, _NW * kc, _CH)
            .transpose(1, 0, 2)
            .reshape(-1))
    n_acc = -(-(n + 1) // 128) * 128
    n_dn = -(-n_acc // 1024) * 128

    jidx = jnp.arange(128) // 16
    seg = (jidx[:, None] == jidx[None, :]).astype(f32)
    x16 = (jnp.arange(16)[:, None] == jidx[None, :]).astype(f32)

    r = 1000
    gn = n // r

    h0 = pl.pallas_call(
        _h0_body,
        grid=(gn,),
        in_specs=[pl.BlockSpec((r, 1), lambda i: (i, 0)),
                  pl.BlockSpec((r, 128), lambda i: (i, 0)),
                  pl.BlockSpec((5, 128), lambda i: (0, 0))],
        out_specs=pl.BlockSpec((r, 128), lambda i: (i, 0)),
        out_shape=jax.ShapeDtypeStruct((n, 128), f32),
    )(node_types.reshape(n, 1), x, nt_emb)

    eb = e_num // 128
    ebs = eb
    nsteps = 1
    tab8 = pl.pallas_call(
        functools.partial(_counts_body, nsteps=nsteps, e_total=float(e_num)),
        grid=(nsteps,),
        in_specs=[pl.BlockSpec((ebs, 128), lambda i: (i, 0)),
                  pl.BlockSpec((3, 128), lambda i: (0, 0))],
        out_specs=pl.BlockSpec((8, 128), lambda i: (0, 0)),
        out_shape=jax.ShapeDtypeStruct((8, 128), f32),
        scratch_shapes=[pltpu.SMEM((3,), f32)],
    )(edge_attr.reshape(eb, 128), et_emb)

    proj_call = pl.pallas_call(
        _proj_body,
        grid=(gn,),
        in_specs=[pl.BlockSpec((r, 128), lambda i: (i, 0)),
                  pl.BlockSpec((128, 128), lambda i: (0, 0)),
                  pl.BlockSpec((1, 128), lambda i: (0, 0)),
                  pl.BlockSpec((128, 128), lambda i: (0, 0)),
                  pl.BlockSpec((1, 128), lambda i: (0, 0))],
        out_specs=[pl.BlockSpec((r, 128), lambda i: (i, 0)),
                   pl.BlockSpec((r, 128), lambda i: (i, 0))],
        out_shape=[jax.ShapeDtypeStruct((n, 128), f32),
                   jax.ShapeDtypeStruct((n, 128), f32)],
    )

    tabproj_call = pl.pallas_call(
        _tabproj_body,
        out_shape=jax.ShapeDtypeStruct((8, 128), f32),
    )

    def epi_call(final, nout):
        extra = ([pl.BlockSpec((128, 128), lambda i: (0, 0)),
                  pl.BlockSpec((1, 128), lambda i: (0, 0))] if final else [])
        return pl.pallas_call(
            functools.partial(_epi_body, final=final),
            grid=(gn,),
            in_specs=[pl.BlockSpec((r, 128), lambda i: (i, 0)),
                      pl.BlockSpec((r, 128), lambda i: (i, 0)),
                      pl.BlockSpec((r, 128), lambda i: (i, 0)),
                      pl.BlockSpec((2, r, 128), lambda i: (0, i, 0)),
                      pl.BlockSpec((2, r, 16), lambda i: (0, i, 0)),
                      pl.BlockSpec((8, 128), lambda i: (0, 0)),
                      pl.BlockSpec((1, 128), lambda i: (0, 0)),
                      pl.BlockSpec((128, 128), lambda i: (0, 0)),
                      pl.BlockSpec((16, 128), lambda i: (0, 0)),
                      pl.BlockSpec((1, 128), lambda i: (0, 0))] + extra,
            out_specs=pl.BlockSpec((r, 128), lambda i: (i, 0)),
            out_shape=jax.ShapeDtypeStruct((nout, 128), f32),
        )

    h = h0
    layers = [(Wl0, bl0, Wr0, br0, We0, att0, bias0),
              (Wl1, bl1, Wr1, br1, We1, att1, bias1)]
    for li, (wl, bl, wr, br, we, att, bias) in enumerate(layers):
        proj8 = tabproj_call(tab8, we)
        xl, xr = proj_call(h, wl, bl.reshape(1, 128), wr, br.reshape(1, 128))
        accp, dnp = _sc_edge_pass(idxp, xl, xr, proj8, att,
                                  kc=kc, n_acc=n_acc, n_dn=n_dn)
        final = li == 1
        args = [h, xl, xr, accp, dnp, proj8, att.reshape(1, 128), seg, x16,
                bias.reshape(1, 128)]
        if final:
            args += [Wo, bo.reshape(1, 128)]
        h = epi_call(final, n)(*args)
    return h


def _sc_edge_pass(idxp, xl, xr, proj8, att, *, kc, n_acc, n_dn):
    f32 = jnp.float32
    mesh = plsc.VectorSubcoreMesh(core_axis_name="c", subcore_axis_name="s",
                                  num_cores=_NC, num_subcores=_NS)
    sc_call = pl.kernel(
        functools.partial(_sc_body, kc=kc, n_acc=n_acc, n_dn=n_dn),
        out_type=(jax.ShapeDtypeStruct((_NC, n_acc, 128), f32),
                  jax.ShapeDtypeStruct((_NC, n_dn, 128), f32)),
        mesh=mesh,
        compiler_params=pltpu.CompilerParams(needs_layout_passes=False),
        scratch_types=[
            pltpu.VMEM((4 * _CH,), jnp.int32),
            pltpu.VMEM((4 * _CH,), jnp.int32),
            pltpu.VMEM((_CH,), jnp.int32),
            pltpu.VMEM((_CH,), jnp.int32),
            pltpu.VMEM((_CH,), jnp.int32),
            pltpu.VMEM((_CH,), jnp.int32),
            pltpu.VMEM((_CH, 128), f32),
            pltpu.VMEM((_CH, 128), f32),
            pltpu.VMEM((_CH, 128), f32),
            pltpu.VMEM((_CH, 128), f32),
            pltpu.VMEM((_CH, 128), f32),
            pltpu.VMEM((_CH, 128), f32),
            pltpu.VMEM((1024,), f32),
            pltpu.VMEM((1, 128), f32),
            pltpu.MemorySpace.VMEM_SHARED((n_acc, 128), f32),
            pltpu.MemorySpace.VMEM_SHARED((n_dn, 128), f32),
        ] + [pltpu.SemaphoreType.DMA] * 8,
    )
    accp, dnp = sc_call(idxp, xl, xr, proj8.reshape(-1), att.reshape(1, 128))
    return accp, dnp.reshape(_NC, n_dn * 8, 16)
